# baseline (speedup 1.0000x reference)
import functools

import jax
import jax.numpy as jnp
from jax import lax
from jax.experimental import pallas as pl
from jax.experimental.pallas import tpu as pltpu
from jax.experimental.pallas import tpu_sc as plsc

NN = 10000
NP = 10240
DD = 256
DH = 128
EE = 160000
EP = 16 * NP
CHUNK = 128
NCHUNKS = EP // CHUNK
NTILES = 16
CPT = NCHUNKS // NTILES
ROWS_PER_TILE = NP // NTILES
RBLK = 1024


def _sc_aggregate(xf, src, dst):
    mesh = plsc.VectorSubcoreMesh(core_axis_name="c", subcore_axis_name="s",
                                  num_cores=2, num_subcores=NTILES)

    @functools.partial(
        pl.kernel,
        mesh=mesh,
        out_type=[
            jax.ShapeDtypeStruct((2 * NP, DH), jnp.float32),
            jax.ShapeDtypeStruct((NTILES, NP), jnp.float32),
        ],
        scratch_types=[
            pltpu.VMEM((CHUNK,), jnp.int32),
            pltpu.VMEM((CHUNK,), jnp.int32),
            pltpu.VMEM((CHUNK, DH), jnp.float32),
            pltpu.VMEM((NP,), jnp.float32),
            pltpu.VMEM_SHARED((NP, DH), jnp.float32),
            pltpu.SemaphoreType.DMA,
        ],
        compiler_params=pltpu.CompilerParams(needs_layout_passes=False),
    )
    def k(xf_hbm, src_hbm, dst_hbm, agg_hbm, degp_hbm,
          src_v, dst_v, rows_v, deg_v, acc_sh, sem):
        c = lax.axis_index("c")
        s = lax.axis_index("s")
        zero16 = jnp.zeros((16,), jnp.float32)
        ones16 = jnp.ones((16,), jnp.float32)
        c_off = c * NP

        def _zrow(i, carry):
            def _zcol(j, carry2):
                rows_v[i, pl.ds(j * 16, 16)] = zero16
                return carry2
            return lax.fori_loop(0, DH // 16, _zcol, carry)
        lax.fori_loop(0, CHUNK, _zrow, 0)

        def _zdeg(i, carry):
            deg_v[pl.ds(i * 16, 16)] = zero16
            return carry
        lax.fori_loop(0, NP // 16, _zdeg, 0)

        base = s * ROWS_PER_TILE
        for b in range(ROWS_PER_TILE // CHUNK):
            pltpu.sync_copy(rows_v,
                            acc_sh.at[pl.ds(base + b * CHUNK, CHUNK)])
        plsc.subcore_barrier()

        def _edge_chunk(t, carry):
            chunk = s + t * NTILES
            off = chunk * CHUNK
            pltpu.sync_copy(src_hbm.at[pl.ds(off, CHUNK)], src_v)
            pltpu.sync_copy(dst_hbm.at[pl.ds(off, CHUNK)], dst_v)
            for j in range(CHUNK // 16):
                sl = pl.ds(j * 16, 16)
                src_v[sl] = src_v[sl] + c_off
            pltpu.async_copy(xf_hbm.at[src_v], rows_v, sem).wait()
            pltpu.sync_copy(rows_v, acc_sh.at[dst_v], add=True)

            @pl.when(c == 0)
            def _():
                for j in range(CHUNK // 16):
                    d16 = dst_v[pl.ds(j * 16, 16)]
                    plsc.addupdate_scatter(deg_v, [d16], ones16)
            return carry

        lax.fori_loop(0, CPT, _edge_chunk, 0)
        plsc.subcore_barrier()

        pltpu.sync_copy(acc_sh.at[pl.ds(base, ROWS_PER_TILE)],
                        agg_hbm.at[pl.ds(c_off + base, ROWS_PER_TILE)])

        @pl.when(c == 0)
        def _():
            pltpu.sync_copy(deg_v, degp_hbm.at[s])

    return k(xf, src, dst)


def _tc_update(aggf, degp, xf, wl2, bl2d, wr2, last):
    nblk = NP // RBLK

    def body(dp_ref, a0_ref, a1_ref, x0_ref, x1_ref, wl_ref, wr_ref, b_ref,
             o_ref):
        deg = jnp.sum(dp_ref[...], axis=0)
        inv = 1.0 / jnp.maximum(deg, 1.0)
        h = ((a0_ref[...] * inv[:, None]) @ wl_ref[0]
             + (a1_ref[...] * inv[:, None]) @ wl_ref[1]
             + x0_ref[...] @ wr_ref[0]
             + x1_ref[...] @ wr_ref[1]
             + b_ref[...])
        if last:
            o_ref[...] = h
        else:
            nrm = jnp.sqrt(jnp.sum(h * h, axis=1, keepdims=True))
            h = h / jnp.maximum(nrm, 1e-12)
            h = jnp.maximum(h, 0.0)
            o_ref[0] = h[:, :DH]
            o_ref[1] = h[:, DH:]

    if last:
        out_shape = jax.ShapeDtypeStruct((NP, DD), jnp.float32)
        out_spec = pl.BlockSpec((RBLK, DD), lambda i: (i, 0))
    else:
        out_shape = jax.ShapeDtypeStruct((2, NP, DH), jnp.float32)
        out_spec = pl.BlockSpec((2, RBLK, DH), lambda i: (0, i, 0))

    return pl.pallas_call(
        body,
        grid=(nblk,),
        in_specs=[
            pl.BlockSpec((NTILES, RBLK), lambda i: (0, i)),
            pl.BlockSpec((RBLK, DH), lambda i: (i, 0)),
            pl.BlockSpec((RBLK, DH), lambda i: (i + nblk, 0)),
            pl.BlockSpec((RBLK, DH), lambda i: (i, 0)),
            pl.BlockSpec((RBLK, DH), lambda i: (i + nblk, 0)),
            pl.BlockSpec((2, DH, DD), lambda i: (0, 0, 0)),
            pl.BlockSpec((2, DH, DD), lambda i: (0, 0, 0)),
            pl.BlockSpec((1, DD), lambda i: (0, 0)),
        ],
        out_specs=out_spec,
        out_shape=out_shape,
        compiler_params=pltpu.CompilerParams(
            dimension_semantics=("arbitrary",)),
    )(degp, aggf, aggf, xf, xf, wl2, wr2, bl2d)


def kernel(x, adjs, Wl0, bl0, Wr0, Wl1, bl1, Wr1, Wl2, bl2, Wr2):
    params = [(Wl0, bl0, Wr0), (Wl1, bl1, Wr1), (Wl2, bl2, Wr2)]
    xp = jnp.pad(x, ((0, NP - NN), (0, 0)))
    xf = xp.reshape(NP, 2, DH).transpose(1, 0, 2).reshape(2 * NP, DH)
    epad = jnp.full((EP - EE,), NN, dtype=jnp.int32)
    out = None
    for i in range(3):
        src = jnp.concatenate([adjs[i, 0, 0], epad])
        dst = jnp.concatenate([adjs[i, 0, 1], epad])
        Wl, bl, Wr = params[i]
        aggf, degp = _sc_aggregate(xf, src, dst)
        wl2 = Wl.reshape(2, DH, DD)
        wr2 = Wr.reshape(2, DH, DD)
        bl2d = bl.reshape(1, DD)
        last = i == 2
        y = _tc_update(aggf, degp, xf, wl2, bl2d, wr2, last)
        if last:
            out = y[:NN]
        else:
            xf = y.reshape(2 * NP, DH)
    return out


# speedup vs baseline: 1.4505x; 1.4505x over previous
import functools

import jax
import jax.numpy as jnp
from jax import lax
from jax.experimental import pallas as pl
from jax.experimental.pallas import tpu as pltpu
from jax.experimental.pallas import tpu_sc as plsc

NN = 10000
NP = 10240
DD = 256
DH = 128
EE = 160000
EP = 16 * NP
CHUNK = 128
NCHUNKS = EP // CHUNK
NTILES = 16
CPT = NCHUNKS // NTILES
ROWS_PER_TILE = NP // NTILES
RBLK = 1024


def _sc_aggregate(xf, src, dst):
    mesh = plsc.VectorSubcoreMesh(core_axis_name="c", subcore_axis_name="s",
                                  num_cores=2, num_subcores=NTILES)

    @functools.partial(
        pl.kernel,
        mesh=mesh,
        out_type=[
            jax.ShapeDtypeStruct((2 * NP, DH), jnp.float32),
            jax.ShapeDtypeStruct((NTILES, NP), jnp.float32),
        ],
        scratch_types=[
            pltpu.VMEM((CHUNK,), jnp.int32),
            pltpu.VMEM((CHUNK,), jnp.int32),
            pltpu.VMEM((CHUNK, DH), jnp.float32),
            pltpu.VMEM((NP,), jnp.float32),
            pltpu.VMEM_SHARED((NP, DH), jnp.float32),
            pltpu.SemaphoreType.DMA,
        ],
        compiler_params=pltpu.CompilerParams(needs_layout_passes=False),
    )
    def k(xf_hbm, src_hbm, dst_hbm, agg_hbm, degp_hbm,
          src_v, dst_v, rows_v, deg_v, acc_sh, sem):
        c = lax.axis_index("c")
        s = lax.axis_index("s")
        zero16 = jnp.zeros((16,), jnp.float32)
        ones16 = jnp.ones((16,), jnp.float32)
        c_off = c * NP

        def _zrow(i, carry):
            def _zcol(j, carry2):
                rows_v[i, pl.ds(j * 16, 16)] = zero16
                return carry2
            return lax.fori_loop(0, DH // 16, _zcol, carry)
        lax.fori_loop(0, CHUNK, _zrow, 0)

        def _zdeg(i, carry):
            deg_v[pl.ds(i * 16, 16)] = zero16
            return carry
        lax.fori_loop(0, NP // 16, _zdeg, 0)

        base = s * ROWS_PER_TILE
        for b in range(ROWS_PER_TILE // CHUNK):
            pltpu.sync_copy(rows_v,
                            acc_sh.at[pl.ds(base + b * CHUNK, CHUNK)])
        plsc.subcore_barrier()

        def _edge_chunk(t, carry):
            chunk = s + t * NTILES
            off = chunk * CHUNK
            pltpu.sync_copy(src_hbm.at[pl.ds(off, CHUNK)], src_v)
            pltpu.sync_copy(dst_hbm.at[pl.ds(off, CHUNK)], dst_v)
            for j in range(CHUNK // 16):
                sl = pl.ds(j * 16, 16)
                src_v[sl] = src_v[sl] + c_off
            pltpu.async_copy(xf_hbm.at[src_v], rows_v, sem).wait()
            pltpu.sync_copy(rows_v, acc_sh.at[dst_v], add=True)

            @pl.when(c == 0)
            def _():
                for j in range(CHUNK // 16):
                    d16 = dst_v[pl.ds(j * 16, 16)]
                    plsc.addupdate_scatter(deg_v, [d16], ones16)
            return carry

        lax.fori_loop(0, CPT, _edge_chunk, 0)
        plsc.subcore_barrier()

        pltpu.sync_copy(acc_sh.at[pl.ds(base, ROWS_PER_TILE)],
                        agg_hbm.at[pl.ds(c_off + base, ROWS_PER_TILE)])

        @pl.when(c == 0)
        def _():
            pltpu.sync_copy(deg_v, degp_hbm.at[s])

    return k(xf, src, dst)


def _tc_update(aggf, degp, xf, wl2, bl2d, wr2, last):
    nblk = NP // RBLK

    def body(dp_ref, a0_ref, a1_ref, x0_ref, x1_ref, wl_ref, wr_ref, b_ref,
             o_ref):
        deg = jnp.sum(dp_ref[...], axis=0)
        inv = 1.0 / jnp.maximum(deg, 1.0)
        h = ((a0_ref[...] * inv[:, None]) @ wl_ref[0]
             + (a1_ref[...] * inv[:, None]) @ wl_ref[1]
             + x0_ref[...] @ wr_ref[0]
             + x1_ref[...] @ wr_ref[1]
             + b_ref[...])
        if last:
            o_ref[...] = h
        else:
            nrm = jnp.sqrt(jnp.sum(h * h, axis=1, keepdims=True))
            h = h / jnp.maximum(nrm, 1e-12)
            h = jnp.maximum(h, 0.0)
            o_ref[0] = h[:, :DH]
            o_ref[1] = h[:, DH:]

    if last:
        out_shape = jax.ShapeDtypeStruct((NP, DD), jnp.float32)
        out_spec = pl.BlockSpec((RBLK, DD), lambda i: (i, 0))
    else:
        out_shape = jax.ShapeDtypeStruct((2, NP, DH), jnp.float32)
        out_spec = pl.BlockSpec((2, RBLK, DH), lambda i: (0, i, 0))

    return pl.pallas_call(
        body,
        grid=(nblk,),
        in_specs=[
            pl.BlockSpec((NTILES, RBLK), lambda i: (0, i)),
            pl.BlockSpec((RBLK, DH), lambda i: (i, 0)),
            pl.BlockSpec((RBLK, DH), lambda i: (i + nblk, 0)),
            pl.BlockSpec((RBLK, DH), lambda i: (i, 0)),
            pl.BlockSpec((RBLK, DH), lambda i: (i + nblk, 0)),
            pl.BlockSpec((2, DH, DD), lambda i: (0, 0, 0)),
            pl.BlockSpec((2, DH, DD), lambda i: (0, 0, 0)),
            pl.BlockSpec((1, DD), lambda i: (0, 0)),
        ],
        out_specs=out_spec,
        out_shape=out_shape,
        compiler_params=pltpu.CompilerParams(
            dimension_semantics=("arbitrary",)),
    )(degp, aggf, aggf, xf, xf, wl2, wr2, bl2d)


def kernel(x, adjs, Wl0, bl0, Wr0, Wl1, bl1, Wr1, Wl2, bl2, Wr2):
    params = [(Wl0, bl0, Wr0), (Wl1, bl1, Wr1), (Wl2, bl2, Wr2)]
    xp = jnp.pad(x, ((0, NP - NN), (0, 0)))
    xf = xp.reshape(NP, 2, DH).transpose(1, 0, 2).reshape(2 * NP, DH)
    epad = NN + (jnp.arange(EP - EE, dtype=jnp.int32) % (NP - NN))
    out = None
    for i in range(3):
        src = jnp.concatenate([adjs[i, 0, 0], epad])
        dst = jnp.concatenate([adjs[i, 0, 1], epad])
        Wl, bl, Wr = params[i]
        aggf, degp = _sc_aggregate(xf, src, dst)
        wl2 = Wl.reshape(2, DH, DD)
        wr2 = Wr.reshape(2, DH, DD)
        bl2d = bl.reshape(1, DD)
        last = i == 2
        y = _tc_update(aggf, degp, xf, wl2, bl2d, wr2, last)
        if last:
            out = y[:NN]
        else:
            xf = y.reshape(2 * NP, DH)
    return out


# speedup vs baseline: 1.8497x; 1.2752x over previous
import functools

import jax
import jax.numpy as jnp
from jax import lax
from jax.experimental import pallas as pl
from jax.experimental.pallas import tpu as pltpu
from jax.experimental.pallas import tpu_sc as plsc

NN = 10000
NP = 10240
DD = 256
DH = 128
EE = 160000
EP = 16 * NP
CHUNK = 256
NCHUNKS = EP // CHUNK
NTILES = 16
CPT = NCHUNKS // NTILES
ROWS_PER_TILE = NP // NTILES
RBLK = 1024


def _sc_aggregate(xf, src, dst):
    mesh = plsc.VectorSubcoreMesh(core_axis_name="c", subcore_axis_name="s",
                                  num_cores=2, num_subcores=NTILES)

    @functools.partial(
        pl.kernel,
        mesh=mesh,
        out_type=[
            jax.ShapeDtypeStruct((2 * NP, DH), jnp.float32),
            jax.ShapeDtypeStruct((NTILES, NP), jnp.float32),
        ],
        scratch_types=[
            pltpu.VMEM((CHUNK,), jnp.int32),
            pltpu.VMEM((CHUNK,), jnp.int32),
            pltpu.VMEM((CHUNK, DH), jnp.float32),
            pltpu.VMEM((NP,), jnp.float32),
            pltpu.VMEM_SHARED((NP, DH), jnp.float32),
            pltpu.SemaphoreType.DMA,
        ],
        compiler_params=pltpu.CompilerParams(needs_layout_passes=False),
    )
    def k(xf_hbm, src_hbm, dst_hbm, agg_hbm, degp_hbm,
          src_v, dst_v, rows_v, deg_v, acc_sh, sem):
        c = lax.axis_index("c")
        s = lax.axis_index("s")
        zero16 = jnp.zeros((16,), jnp.float32)
        ones16 = jnp.ones((16,), jnp.float32)
        c_off = c * NP

        def _zrow(i, carry):
            def _zcol(j, carry2):
                rows_v[i, pl.ds(j * 16, 16)] = zero16
                return carry2
            return lax.fori_loop(0, DH // 16, _zcol, carry)
        lax.fori_loop(0, CHUNK, _zrow, 0)

        def _zdeg(i, carry):
            deg_v[pl.ds(i * 16, 16)] = zero16
            return carry
        lax.fori_loop(0, NP // 16, _zdeg, 0)

        base = s * ROWS_PER_TILE
        for b in range(ROWS_PER_TILE // CHUNK):
            pltpu.sync_copy(rows_v,
                            acc_sh.at[pl.ds(base + b * CHUNK, CHUNK)])
        if ROWS_PER_TILE % CHUNK:
            pltpu.sync_copy(rows_v.at[pl.ds(0, ROWS_PER_TILE % CHUNK)],
                            acc_sh.at[pl.ds(base + ROWS_PER_TILE - ROWS_PER_TILE % CHUNK,
                                            ROWS_PER_TILE % CHUNK)])
        plsc.subcore_barrier()

        def _edge_chunk(t, carry):
            chunk = s + t * NTILES
            off = chunk * CHUNK
            pltpu.sync_copy(src_hbm.at[pl.ds(off, CHUNK)], src_v)
            pltpu.sync_copy(dst_hbm.at[pl.ds(off, CHUNK)], dst_v)
            for j in range(CHUNK // 16):
                sl = pl.ds(j * 16, 16)
                src_v[sl] = src_v[sl] + c_off
            pltpu.async_copy(xf_hbm.at[src_v], rows_v, sem).wait()
            pltpu.sync_copy(rows_v, acc_sh.at[dst_v], add=True)

            @pl.when(c == 0)
            def _():
                for j in range(CHUNK // 16):
                    d16 = dst_v[pl.ds(j * 16, 16)]
                    plsc.addupdate_scatter(deg_v, [d16], ones16)
            return carry

        lax.fori_loop(0, CPT, _edge_chunk, 0)
        plsc.subcore_barrier()

        pltpu.sync_copy(acc_sh.at[pl.ds(base, ROWS_PER_TILE)],
                        agg_hbm.at[pl.ds(c_off + base, ROWS_PER_TILE)])

        @pl.when(c == 0)
        def _():
            pltpu.sync_copy(deg_v, degp_hbm.at[s])

    return k(xf, src, dst)


def _tc_update(aggf, degp, xf, wl2, bl2d, wr2, last):
    nblk = NP // RBLK

    def body(dp_ref, a0_ref, a1_ref, x0_ref, x1_ref, wl_ref, wr_ref, b_ref,
             o_ref):
        deg = jnp.sum(dp_ref[...], axis=0)
        inv = 1.0 / jnp.maximum(deg, 1.0)
        h = ((a0_ref[...] * inv[:, None]) @ wl_ref[0]
             + (a1_ref[...] * inv[:, None]) @ wl_ref[1]
             + x0_ref[...] @ wr_ref[0]
             + x1_ref[...] @ wr_ref[1]
             + b_ref[...])
        if last:
            o_ref[...] = h
        else:
            nrm = jnp.sqrt(jnp.sum(h * h, axis=1, keepdims=True))
            h = h / jnp.maximum(nrm, 1e-12)
            h = jnp.maximum(h, 0.0)
            o_ref[0] = h[:, :DH]
            o_ref[1] = h[:, DH:]

    if last:
        out_shape = jax.ShapeDtypeStruct((NP, DD), jnp.float32)
        out_spec = pl.BlockSpec((RBLK, DD), lambda i: (i, 0))
    else:
        out_shape = jax.ShapeDtypeStruct((2, NP, DH), jnp.float32)
        out_spec = pl.BlockSpec((2, RBLK, DH), lambda i: (0, i, 0))

    return pl.pallas_call(
        body,
        grid=(nblk,),
        in_specs=[
            pl.BlockSpec((NTILES, RBLK), lambda i: (0, i)),
            pl.BlockSpec((RBLK, DH), lambda i: (i, 0)),
            pl.BlockSpec((RBLK, DH), lambda i: (i + nblk, 0)),
            pl.BlockSpec((RBLK, DH), lambda i: (i, 0)),
            pl.BlockSpec((RBLK, DH), lambda i: (i + nblk, 0)),
            pl.BlockSpec((2, DH, DD), lambda i: (0, 0, 0)),
            pl.BlockSpec((2, DH, DD), lambda i: (0, 0, 0)),
            pl.BlockSpec((1, DD), lambda i: (0, 0)),
        ],
        out_specs=out_spec,
        out_shape=out_shape,
        compiler_params=pltpu.CompilerParams(
            dimension_semantics=("arbitrary",)),
    )(degp, aggf, aggf, xf, xf, wl2, wr2, bl2d)


def kernel(x, adjs, Wl0, bl0, Wr0, Wl1, bl1, Wr1, Wl2, bl2, Wr2):
    params = [(Wl0, bl0, Wr0), (Wl1, bl1, Wr1), (Wl2, bl2, Wr2)]
    xp = jnp.pad(x, ((0, NP - NN), (0, 0)))
    xf = xp.reshape(NP, 2, DH).transpose(1, 0, 2).reshape(2 * NP, DH)
    epad = NN + (jnp.arange(EP - EE, dtype=jnp.int32) % (NP - NN))
    out = None
    for i in range(3):
        src = jnp.concatenate([adjs[i, 0, 0], epad])
        dst = jnp.concatenate([adjs[i, 0, 1], epad])
        Wl, bl, Wr = params[i]
        aggf, degp = _sc_aggregate(xf, src, dst)
        wl2 = Wl.reshape(2, DH, DD)
        wr2 = Wr.reshape(2, DH, DD)
        bl2d = bl.reshape(1, DD)
        last = i == 2
        y = _tc_update(aggf, degp, xf, wl2, bl2d, wr2, last)
        if last:
            out = y[:NN]
        else:
            xf = y.reshape(2 * NP, DH)
    return out


# speedup vs baseline: 2.2038x; 1.1915x over previous
import functools

import jax
import jax.numpy as jnp
from jax import lax
from jax.experimental import pallas as pl
from jax.experimental.pallas import tpu as pltpu
from jax.experimental.pallas import tpu_sc as plsc

NN = 10000
NP = 10240
DD = 256
DH = 128
EE = 160000
EP = 16 * NP
CHUNK = 256
NCHUNKS = EP // CHUNK
NTILES = 16
CPT = NCHUNKS // NTILES
ROWS_PER_TILE = NP // NTILES
RBLK = 1024


def _sc_aggregate(xf, src, dst):
    mesh = plsc.VectorSubcoreMesh(core_axis_name="c", subcore_axis_name="s",
                                  num_cores=2, num_subcores=NTILES)

    @functools.partial(
        pl.kernel,
        mesh=mesh,
        out_type=[
            jax.ShapeDtypeStruct((2 * NP, DH), jnp.float32),
            jax.ShapeDtypeStruct((NTILES, NP), jnp.float32),
        ],
        scratch_types=[
            pltpu.VMEM((CHUNK,), jnp.int32),
            pltpu.VMEM((CHUNK,), jnp.int32),
            pltpu.VMEM((CHUNK,), jnp.int32),
            pltpu.VMEM((CHUNK,), jnp.int32),
            pltpu.VMEM((CHUNK, DH), jnp.float32),
            pltpu.VMEM((NP,), jnp.float32),
            pltpu.VMEM_SHARED((NP, DH), jnp.float32),
            pltpu.SemaphoreType.DMA,
            pltpu.SemaphoreType.DMA,
        ],
        compiler_params=pltpu.CompilerParams(needs_layout_passes=False),
    )
    def k(xf_hbm, src_hbm, dst_hbm, agg_hbm, degp_hbm,
          src_a, dst_a, src_b, dst_b, rows_v, deg_v, acc_sh, sem, semi):
        c = lax.axis_index("c")
        s = lax.axis_index("s")
        zero16 = jnp.zeros((16,), jnp.float32)
        ones16 = jnp.ones((16,), jnp.float32)
        c_off = c * NP

        def _zrow(i, carry):
            def _zcol(j, carry2):
                rows_v[i, pl.ds(j * 16, 16)] = zero16
                return carry2
            return lax.fori_loop(0, DH // 16, _zcol, carry)
        lax.fori_loop(0, CHUNK, _zrow, 0)

        def _zdeg(i, carry):
            deg_v[pl.ds(i * 16, 16)] = zero16
            return carry
        lax.fori_loop(0, NP // 16, _zdeg, 0)

        base = s * ROWS_PER_TILE
        for b in range(ROWS_PER_TILE // CHUNK):
            pltpu.sync_copy(rows_v,
                            acc_sh.at[pl.ds(base + b * CHUNK, CHUNK)])
        if ROWS_PER_TILE % CHUNK:
            pltpu.sync_copy(rows_v.at[pl.ds(0, ROWS_PER_TILE % CHUNK)],
                            acc_sh.at[pl.ds(base + ROWS_PER_TILE - ROWS_PER_TILE % CHUNK,
                                            ROWS_PER_TILE % CHUNK)])
        pltpu.sync_copy(src_hbm.at[pl.ds(s * CHUNK, CHUNK)], src_a)
        pltpu.sync_copy(dst_hbm.at[pl.ds(s * CHUNK, CHUNK)], dst_a)
        plsc.subcore_barrier()

        def _process(sv, dv):
            for j in range(CHUNK // 16):
                sl = pl.ds(j * 16, 16)
                sv[sl] = sv[sl] + c_off
            pltpu.async_copy(xf_hbm.at[sv], rows_v, sem).wait()
            pltpu.sync_copy(rows_v, acc_sh.at[dv], add=True)

            @pl.when(c == 0)
            def _():
                for j in range(CHUNK // 16):
                    d16 = dv[pl.ds(j * 16, 16)]
                    plsc.addupdate_scatter(deg_v, [d16], ones16)

        def _edge_pair(u, carry):
            tb_off = (s + (2 * u + 1) * NTILES) * CHUNK
            dsb = pltpu.async_copy(src_hbm.at[pl.ds(tb_off, CHUNK)],
                                   src_b, semi)
            ddb = pltpu.async_copy(dst_hbm.at[pl.ds(tb_off, CHUNK)],
                                   dst_b, semi)
            _process(src_a, dst_a)
            dsb.wait()
            ddb.wait()
            unext = jnp.minimum(2 * u + 2, CPT - 1)
            ta_off = (s + unext * NTILES) * CHUNK
            dsa = pltpu.async_copy(src_hbm.at[pl.ds(ta_off, CHUNK)],
                                   src_a, semi)
            dda = pltpu.async_copy(dst_hbm.at[pl.ds(ta_off, CHUNK)],
                                   dst_a, semi)
            _process(src_b, dst_b)
            dsa.wait()
            dda.wait()
            return carry

        lax.fori_loop(0, CPT // 2, _edge_pair, 0)
        plsc.subcore_barrier()

        pltpu.sync_copy(acc_sh.at[pl.ds(base, ROWS_PER_TILE)],
                        agg_hbm.at[pl.ds(c_off + base, ROWS_PER_TILE)])

        @pl.when(c == 0)
        def _():
            pltpu.sync_copy(deg_v, degp_hbm.at[s])

    return k(xf, src, dst)


def _tc_update(aggf, degp, xf, wl2, bl2d, wr2, last):
    nblk = NP // RBLK

    def body(dp_ref, a0_ref, a1_ref, x0_ref, x1_ref, wl_ref, wr_ref, b_ref,
             o_ref):
        deg = jnp.sum(dp_ref[...], axis=0)
        inv = 1.0 / jnp.maximum(deg, 1.0)
        h = ((a0_ref[...] * inv[:, None]) @ wl_ref[0]
             + (a1_ref[...] * inv[:, None]) @ wl_ref[1]
             + x0_ref[...] @ wr_ref[0]
             + x1_ref[...] @ wr_ref[1]
             + b_ref[...])
        if last:
            o_ref[...] = h
        else:
            nrm = jnp.sqrt(jnp.sum(h * h, axis=1, keepdims=True))
            h = h / jnp.maximum(nrm, 1e-12)
            h = jnp.maximum(h, 0.0)
            o_ref[0] = h[:, :DH]
            o_ref[1] = h[:, DH:]

    if last:
        out_shape = jax.ShapeDtypeStruct((NP, DD), jnp.float32)
        out_spec = pl.BlockSpec((RBLK, DD), lambda i: (i, 0))
    else:
        out_shape = jax.ShapeDtypeStruct((2, NP, DH), jnp.float32)
        out_spec = pl.BlockSpec((2, RBLK, DH), lambda i: (0, i, 0))

    return pl.pallas_call(
        body,
        grid=(nblk,),
        in_specs=[
            pl.BlockSpec((NTILES, RBLK), lambda i: (0, i)),
            pl.BlockSpec((RBLK, DH), lambda i: (i, 0)),
            pl.BlockSpec((RBLK, DH), lambda i: (i + nblk, 0)),
            pl.BlockSpec((RBLK, DH), lambda i: (i, 0)),
            pl.BlockSpec((RBLK, DH), lambda i: (i + nblk, 0)),
            pl.BlockSpec((2, DH, DD), lambda i: (0, 0, 0)),
            pl.BlockSpec((2, DH, DD), lambda i: (0, 0, 0)),
            pl.BlockSpec((1, DD), lambda i: (0, 0)),
        ],
        out_specs=out_spec,
        out_shape=out_shape,
        compiler_params=pltpu.CompilerParams(
            dimension_semantics=("arbitrary",)),
    )(degp, aggf, aggf, xf, xf, wl2, wr2, bl2d)


def kernel(x, adjs, Wl0, bl0, Wr0, Wl1, bl1, Wr1, Wl2, bl2, Wr2):
    params = [(Wl0, bl0, Wr0), (Wl1, bl1, Wr1), (Wl2, bl2, Wr2)]
    xp = jnp.pad(x, ((0, NP - NN), (0, 0)))
    xf = xp.reshape(NP, 2, DH).transpose(1, 0, 2).reshape(2 * NP, DH)
    epad = NN + (jnp.arange(EP - EE, dtype=jnp.int32) % (NP - NN))
    out = None
    for i in range(3):
        src = jnp.concatenate([adjs[i, 0, 0], epad])
        dst = jnp.concatenate([adjs[i, 0, 1], epad])
        Wl, bl, Wr = params[i]
        aggf, degp = _sc_aggregate(xf, src, dst)
        wl2 = Wl.reshape(2, DH, DD)
        wr2 = Wr.reshape(2, DH, DD)
        bl2d = bl.reshape(1, DD)
        last = i == 2
        y = _tc_update(aggf, degp, xf, wl2, bl2d, wr2, last)
        if last:
            out = y[:NN]
        else:
            xf = y.reshape(2 * NP, DH)
    return out


# speedup vs baseline: 2.2650x; 1.0278x over previous
import functools

import jax
import jax.numpy as jnp
from jax import lax
from jax.experimental import pallas as pl
from jax.experimental.pallas import tpu as pltpu
from jax.experimental.pallas import tpu_sc as plsc

NN = 10000
NP = 10240
DD = 256
DH = 128
EE = 160000
EP = 16 * NP
CHUNK = 256
NCHUNKS = EP // CHUNK
NTILES = 16
CPT = NCHUNKS // NTILES
ROWS_PER_TILE = NP // NTILES
RBLK = 1024


def _sc_aggregate(xf, src, dst):
    mesh = plsc.VectorSubcoreMesh(core_axis_name="c", subcore_axis_name="s",
                                  num_cores=2, num_subcores=NTILES)

    @functools.partial(
        pl.kernel,
        mesh=mesh,
        out_type=[
            jax.ShapeDtypeStruct((2 * NP, DH), jnp.float32),
            jax.ShapeDtypeStruct((NTILES, NP), jnp.float32),
        ],
        scratch_types=[
            pltpu.VMEM((CHUNK // 2,), jnp.int32),
            pltpu.VMEM((CHUNK // 2,), jnp.int32),
            pltpu.VMEM((CHUNK // 2,), jnp.int32),
            pltpu.VMEM((CHUNK // 2,), jnp.int32),
            pltpu.VMEM((CHUNK // 2,), jnp.int32),
            pltpu.VMEM((CHUNK // 2,), jnp.int32),
            pltpu.VMEM((CHUNK // 2,), jnp.int32),
            pltpu.VMEM((CHUNK // 2,), jnp.int32),
            pltpu.VMEM((CHUNK, DH), jnp.float32),
            pltpu.VMEM((NP,), jnp.float32),
            pltpu.VMEM_SHARED((NP, DH), jnp.float32),
            pltpu.SemaphoreType.DMA,
            pltpu.SemaphoreType.DMA,
            pltpu.SemaphoreType.DMA,
            pltpu.SemaphoreType.DMA,
        ],
        compiler_params=pltpu.CompilerParams(needs_layout_passes=False),
    )
    def k(xf_hbm, src_hbm, dst_hbm, agg_hbm, degp_hbm,
          src_a1, src_a2, dst_a1, dst_a2, src_b1, src_b2, dst_b1, dst_b2,
          rows_v, deg_v, acc_sh, sem, sem2, sems, semi):
        c = lax.axis_index("c")
        s = lax.axis_index("s")
        zero16 = jnp.zeros((16,), jnp.float32)
        ones16 = jnp.ones((16,), jnp.float32)
        c_off = c * NP

        def _zrow(i, carry):
            def _zcol(j, carry2):
                rows_v[i, pl.ds(j * 16, 16)] = zero16
                return carry2
            return lax.fori_loop(0, DH // 16, _zcol, carry)
        lax.fori_loop(0, CHUNK, _zrow, 0)

        def _zdeg(i, carry):
            deg_v[pl.ds(i * 16, 16)] = zero16
            return carry
        lax.fori_loop(0, NP // 16, _zdeg, 0)

        base = s * ROWS_PER_TILE
        for b in range(ROWS_PER_TILE // CHUNK):
            pltpu.sync_copy(rows_v,
                            acc_sh.at[pl.ds(base + b * CHUNK, CHUNK)])
        if ROWS_PER_TILE % CHUNK:
            pltpu.sync_copy(rows_v.at[pl.ds(0, ROWS_PER_TILE % CHUNK)],
                            acc_sh.at[pl.ds(base + ROWS_PER_TILE - ROWS_PER_TILE % CHUNK,
                                            ROWS_PER_TILE % CHUNK)])
        HC = CHUNK // 2

        def _load_idx(off, sv1, sv2, dv1, dv2):
            return (pltpu.async_copy(src_hbm.at[pl.ds(off, HC)], sv1, semi),
                    pltpu.async_copy(src_hbm.at[pl.ds(off + HC, HC)], sv2,
                                     semi),
                    pltpu.async_copy(dst_hbm.at[pl.ds(off, HC)], dv1, semi),
                    pltpu.async_copy(dst_hbm.at[pl.ds(off + HC, HC)], dv2,
                                     semi))

        for d in _load_idx(s * CHUNK, src_a1, src_a2, dst_a1, dst_a2):
            d.wait()
        plsc.subcore_barrier()

        def _process(sv1, sv2, dv1, dv2):
            for sv in (sv1, sv2):
                for j in range(HC // 16):
                    sl = pl.ds(j * 16, 16)
                    sv[sl] = sv[sl] + c_off
            dg1 = pltpu.async_copy(xf_hbm.at[sv1], rows_v.at[pl.ds(0, HC)],
                                   sem)
            dg2 = pltpu.async_copy(xf_hbm.at[sv2], rows_v.at[pl.ds(HC, HC)],
                                   sem2)
            dg1.wait()
            ds1 = pltpu.async_copy(rows_v.at[pl.ds(0, HC)], acc_sh.at[dv1],
                                   sems, add=True)
            dg2.wait()
            ds2 = pltpu.async_copy(rows_v.at[pl.ds(HC, HC)], acc_sh.at[dv2],
                                   sems, add=True)

            @pl.when(c == 0)
            def _():
                for dv in (dv1, dv2):
                    for j in range(HC // 16):
                        d16 = dv[pl.ds(j * 16, 16)]
                        plsc.addupdate_scatter(deg_v, [d16], ones16)
            ds1.wait()
            ds2.wait()

        def _edge_pair(u, carry):
            tb_off = (s + (2 * u + 1) * NTILES) * CHUNK
            db = _load_idx(tb_off, src_b1, src_b2, dst_b1, dst_b2)
            _process(src_a1, src_a2, dst_a1, dst_a2)
            for d in db:
                d.wait()
            unext = jnp.minimum(2 * u + 2, CPT - 1)
            ta_off = (s + unext * NTILES) * CHUNK
            da = _load_idx(ta_off, src_a1, src_a2, dst_a1, dst_a2)
            _process(src_b1, src_b2, dst_b1, dst_b2)
            for d in da:
                d.wait()
            return carry

        lax.fori_loop(0, CPT // 2, _edge_pair, 0)
        plsc.subcore_barrier()

        pltpu.sync_copy(acc_sh.at[pl.ds(base, ROWS_PER_TILE)],
                        agg_hbm.at[pl.ds(c_off + base, ROWS_PER_TILE)])

        @pl.when(c == 0)
        def _():
            pltpu.sync_copy(deg_v, degp_hbm.at[s])

    return k(xf, src, dst)


def _tc_update(aggf, degp, xf, wl2, bl2d, wr2, last):
    nblk = NP // RBLK

    def body(dp_ref, a0_ref, a1_ref, x0_ref, x1_ref, wl_ref, wr_ref, b_ref,
             o_ref):
        deg = jnp.sum(dp_ref[...], axis=0)
        inv = 1.0 / jnp.maximum(deg, 1.0)
        h = ((a0_ref[...] * inv[:, None]) @ wl_ref[0]
             + (a1_ref[...] * inv[:, None]) @ wl_ref[1]
             + x0_ref[...] @ wr_ref[0]
             + x1_ref[...] @ wr_ref[1]
             + b_ref[...])
        if last:
            o_ref[...] = h
        else:
            nrm = jnp.sqrt(jnp.sum(h * h, axis=1, keepdims=True))
            h = h / jnp.maximum(nrm, 1e-12)
            h = jnp.maximum(h, 0.0)
            o_ref[0] = h[:, :DH]
            o_ref[1] = h[:, DH:]

    if last:
        out_shape = jax.ShapeDtypeStruct((NP, DD), jnp.float32)
        out_spec = pl.BlockSpec((RBLK, DD), lambda i: (i, 0))
    else:
        out_shape = jax.ShapeDtypeStruct((2, NP, DH), jnp.float32)
        out_spec = pl.BlockSpec((2, RBLK, DH), lambda i: (0, i, 0))

    return pl.pallas_call(
        body,
        grid=(nblk,),
        in_specs=[
            pl.BlockSpec((NTILES, RBLK), lambda i: (0, i)),
            pl.BlockSpec((RBLK, DH), lambda i: (i, 0)),
            pl.BlockSpec((RBLK, DH), lambda i: (i + nblk, 0)),
            pl.BlockSpec((RBLK, DH), lambda i: (i, 0)),
            pl.BlockSpec((RBLK, DH), lambda i: (i + nblk, 0)),
            pl.BlockSpec((2, DH, DD), lambda i: (0, 0, 0)),
            pl.BlockSpec((2, DH, DD), lambda i: (0, 0, 0)),
            pl.BlockSpec((1, DD), lambda i: (0, 0)),
        ],
        out_specs=out_spec,
        out_shape=out_shape,
        compiler_params=pltpu.CompilerParams(
            dimension_semantics=("arbitrary",)),
    )(degp, aggf, aggf, xf, xf, wl2, wr2, bl2d)


def kernel(x, adjs, Wl0, bl0, Wr0, Wl1, bl1, Wr1, Wl2, bl2, Wr2):
    params = [(Wl0, bl0, Wr0), (Wl1, bl1, Wr1), (Wl2, bl2, Wr2)]
    xp = jnp.pad(x, ((0, NP - NN), (0, 0)))
    xf = xp.reshape(NP, 2, DH).transpose(1, 0, 2).reshape(2 * NP, DH)
    epad = NN + (jnp.arange(EP - EE, dtype=jnp.int32) % (NP - NN))
    out = None
    for i in range(3):
        src = jnp.concatenate([adjs[i, 0, 0], epad])
        dst = jnp.concatenate([adjs[i, 0, 1], epad])
        Wl, bl, Wr = params[i]
        aggf, degp = _sc_aggregate(xf, src, dst)
        wl2 = Wl.reshape(2, DH, DD)
        wr2 = Wr.reshape(2, DH, DD)
        bl2d = bl.reshape(1, DD)
        last = i == 2
        y = _tc_update(aggf, degp, xf, wl2, bl2d, wr2, last)
        if last:
            out = y[:NN]
        else:
            xf = y.reshape(2 * NP, DH)
    return out


# speedup vs baseline: 2.2770x; 1.0053x over previous
import functools

import jax
import jax.numpy as jnp
from jax import lax
from jax.experimental import pallas as pl
from jax.experimental.pallas import tpu as pltpu
from jax.experimental.pallas import tpu_sc as plsc

NN = 10000
NP = 10240
DD = 256
DH = 128
EE = 160000
EP = 16 * NP
CHUNK = 256
NCHUNKS = EP // CHUNK
NTILES = 16
CPT = NCHUNKS // NTILES
ROWS_PER_TILE = NP // NTILES
RBLK = 1024


def _sc_aggregate(xf, src, dst):
    mesh = plsc.VectorSubcoreMesh(core_axis_name="c", subcore_axis_name="s",
                                  num_cores=2, num_subcores=NTILES)

    @functools.partial(
        pl.kernel,
        mesh=mesh,
        out_type=[
            jax.ShapeDtypeStruct((2 * NP, DH), jnp.float32),
            jax.ShapeDtypeStruct((NTILES, NP), jnp.float32),
        ],
        scratch_types=[
            pltpu.VMEM((CHUNK // 2,), jnp.int32),
            pltpu.VMEM((CHUNK // 2,), jnp.int32),
            pltpu.VMEM((CHUNK // 2,), jnp.int32),
            pltpu.VMEM((CHUNK // 2,), jnp.int32),
            pltpu.VMEM((CHUNK // 2,), jnp.int32),
            pltpu.VMEM((CHUNK // 2,), jnp.int32),
            pltpu.VMEM((CHUNK // 2,), jnp.int32),
            pltpu.VMEM((CHUNK // 2,), jnp.int32),
            pltpu.VMEM((CHUNK, DH), jnp.float32),
            pltpu.VMEM((NP,), jnp.float32),
            pltpu.VMEM_SHARED((NP, DH), jnp.float32),
            pltpu.SemaphoreType.DMA,
            pltpu.SemaphoreType.DMA,
            pltpu.SemaphoreType.DMA,
            pltpu.SemaphoreType.DMA,
        ],
        compiler_params=pltpu.CompilerParams(needs_layout_passes=False),
    )
    def k(xf_hbm, src_hbm, dst_hbm, agg_hbm, degp_hbm,
          src_a1, src_a2, dst_a1, dst_a2, src_b1, src_b2, dst_b1, dst_b2,
          rows_v, deg_v, acc_sh, sem, sem2, sems, semi):
        c = lax.axis_index("c")
        s = lax.axis_index("s")
        zero16 = jnp.zeros((16,), jnp.float32)
        ones16 = jnp.ones((16,), jnp.float32)
        c_off = c * NP

        def _zrow(i, carry):
            def _zcol(j, carry2):
                rows_v[i, pl.ds(j * 16, 16)] = zero16
                return carry2
            return lax.fori_loop(0, DH // 16, _zcol, carry)
        lax.fori_loop(0, CHUNK, _zrow, 0)

        def _zdeg(i, carry):
            deg_v[pl.ds(i * 16, 16)] = zero16
            return carry
        lax.fori_loop(0, NP // 16, _zdeg, 0)

        base = s * ROWS_PER_TILE
        for b in range(ROWS_PER_TILE // CHUNK):
            pltpu.sync_copy(rows_v,
                            acc_sh.at[pl.ds(base + b * CHUNK, CHUNK)])
        if ROWS_PER_TILE % CHUNK:
            pltpu.sync_copy(rows_v.at[pl.ds(0, ROWS_PER_TILE % CHUNK)],
                            acc_sh.at[pl.ds(base + ROWS_PER_TILE - ROWS_PER_TILE % CHUNK,
                                            ROWS_PER_TILE % CHUNK)])
        HC = CHUNK // 2

        def _load_idx(off, sv1, sv2, dv1, dv2):
            return (pltpu.async_copy(src_hbm.at[pl.ds(off, HC)], sv1, semi),
                    pltpu.async_copy(src_hbm.at[pl.ds(off + HC, HC)], sv2,
                                     semi),
                    pltpu.async_copy(dst_hbm.at[pl.ds(off, HC)], dv1, semi),
                    pltpu.async_copy(dst_hbm.at[pl.ds(off + HC, HC)], dv2,
                                     semi))

        for d in _load_idx(s * CHUNK, src_a1, src_a2, dst_a1, dst_a2):
            d.wait()
        plsc.subcore_barrier()

        def _process(sv1, sv2, dv1, dv2):
            for sv in (sv1, sv2):
                for j in range(HC // 16):
                    sl = pl.ds(j * 16, 16)
                    sv[sl] = sv[sl] + c_off
            dg1 = pltpu.async_copy(xf_hbm.at[sv1], rows_v.at[pl.ds(0, HC)],
                                   sem)
            dg2 = pltpu.async_copy(xf_hbm.at[sv2], rows_v.at[pl.ds(HC, HC)],
                                   sem2)
            dg1.wait()
            ds1 = pltpu.async_copy(rows_v.at[pl.ds(0, HC)], acc_sh.at[dv1],
                                   sems, add=True)
            dg2.wait()
            ds2 = pltpu.async_copy(rows_v.at[pl.ds(HC, HC)], acc_sh.at[dv2],
                                   sems, add=True)

            @pl.when(c == 0)
            def _():
                for dv in (dv1, dv2):
                    for j in range(HC // 16):
                        d16 = dv[pl.ds(j * 16, 16)]
                        plsc.addupdate_scatter(deg_v, [d16], ones16)
            ds1.wait()
            ds2.wait()

        def _edge_pair(u, carry):
            tb_off = (s + (2 * u + 1) * NTILES) * CHUNK
            db = _load_idx(tb_off, src_b1, src_b2, dst_b1, dst_b2)
            _process(src_a1, src_a2, dst_a1, dst_a2)
            for d in db:
                d.wait()
            unext = jnp.minimum(2 * u + 2, CPT - 1)
            ta_off = (s + unext * NTILES) * CHUNK
            da = _load_idx(ta_off, src_a1, src_a2, dst_a1, dst_a2)
            _process(src_b1, src_b2, dst_b1, dst_b2)
            for d in da:
                d.wait()
            return carry

        lax.fori_loop(0, CPT // 2, _edge_pair, 0)
        plsc.subcore_barrier()

        pltpu.sync_copy(acc_sh.at[pl.ds(base, ROWS_PER_TILE)],
                        agg_hbm.at[pl.ds(c_off + base, ROWS_PER_TILE)])

        @pl.when(c == 0)
        def _():
            pltpu.sync_copy(deg_v, degp_hbm.at[s])

    return k(xf, src, dst)


def _tc_residual(xf, wr2, bl2d):
    nblk = NP // RBLK

    def body(x0_ref, x1_ref, wr_ref, b_ref, o_ref):
        o_ref[...] = (x0_ref[...] @ wr_ref[0]
                      + x1_ref[...] @ wr_ref[1]
                      + b_ref[...])

    return pl.pallas_call(
        body,
        grid=(nblk,),
        in_specs=[
            pl.BlockSpec((RBLK, DH), lambda i: (i, 0)),
            pl.BlockSpec((RBLK, DH), lambda i: (i + nblk, 0)),
            pl.BlockSpec((2, DH, DD), lambda i: (0, 0, 0)),
            pl.BlockSpec((1, DD), lambda i: (0, 0)),
        ],
        out_specs=pl.BlockSpec((RBLK, DD), lambda i: (i, 0)),
        out_shape=jax.ShapeDtypeStruct((NP, DD), jnp.float32),
        compiler_params=pltpu.CompilerParams(
            dimension_semantics=("arbitrary",)),
    )(xf, xf, wr2, bl2d)


def _tc_combine(aggf, degp, res, wl2, last):
    nblk = NP // RBLK

    def body(dp_ref, a0_ref, a1_ref, r_ref, wl_ref, o_ref):
        deg = jnp.sum(dp_ref[...], axis=0)
        inv = 1.0 / jnp.maximum(deg, 1.0)
        h = ((a0_ref[...] * inv[:, None]) @ wl_ref[0]
             + (a1_ref[...] * inv[:, None]) @ wl_ref[1]
             + r_ref[...])
        if last:
            o_ref[...] = h
        else:
            nrm = jnp.sqrt(jnp.sum(h * h, axis=1, keepdims=True))
            h = h / jnp.maximum(nrm, 1e-12)
            h = jnp.maximum(h, 0.0)
            o_ref[0] = h[:, :DH]
            o_ref[1] = h[:, DH:]

    if last:
        out_shape = jax.ShapeDtypeStruct((NP, DD), jnp.float32)
        out_spec = pl.BlockSpec((RBLK, DD), lambda i: (i, 0))
    else:
        out_shape = jax.ShapeDtypeStruct((2, NP, DH), jnp.float32)
        out_spec = pl.BlockSpec((2, RBLK, DH), lambda i: (0, i, 0))

    return pl.pallas_call(
        body,
        grid=(nblk,),
        in_specs=[
            pl.BlockSpec((NTILES, RBLK), lambda i: (0, i)),
            pl.BlockSpec((RBLK, DH), lambda i: (i, 0)),
            pl.BlockSpec((RBLK, DH), lambda i: (i + nblk, 0)),
            pl.BlockSpec((RBLK, DD), lambda i: (i, 0)),
            pl.BlockSpec((2, DH, DD), lambda i: (0, 0, 0)),
        ],
        out_specs=out_spec,
        out_shape=out_shape,
        compiler_params=pltpu.CompilerParams(
            dimension_semantics=("arbitrary",)),
    )(degp, aggf, aggf, res, wl2)


def kernel(x, adjs, Wl0, bl0, Wr0, Wl1, bl1, Wr1, Wl2, bl2, Wr2):
    params = [(Wl0, bl0, Wr0), (Wl1, bl1, Wr1), (Wl2, bl2, Wr2)]
    xp = jnp.pad(x, ((0, NP - NN), (0, 0)))
    xf = xp.reshape(NP, 2, DH).transpose(1, 0, 2).reshape(2 * NP, DH)
    epad = NN + (jnp.arange(EP - EE, dtype=jnp.int32) % (NP - NN))
    out = None
    for i in range(3):
        src = jnp.concatenate([adjs[i, 0, 0], epad])
        dst = jnp.concatenate([adjs[i, 0, 1], epad])
        Wl, bl, Wr = params[i]
        wl2 = Wl.reshape(2, DH, DD)
        wr2 = Wr.reshape(2, DH, DD)
        bl2d = bl.reshape(1, DD)
        aggf, degp = _sc_aggregate(xf, src, dst)
        res = _tc_residual(xf, wr2, bl2d)
        last = i == 2
        y = _tc_combine(aggf, degp, res, wl2, last)
        if last:
            out = y[:NN]
        else:
            xf = y.reshape(2 * NP, DH)
    return out
